# baseline (speedup 1.0000x reference)
import functools

import jax
import jax.numpy as jnp
from jax import lax
from jax.experimental import pallas as pl
from jax.experimental.pallas import tpu as pltpu
from jax.experimental.pallas import tpu_sc as plsc


_TILE = 4096
_NC, _NS = 2, 16
_CHUNK = 128
_DPAD = 128


def _score_body(z_ref, cb_ref, idx_ref, cbn_ref):
    cb = cb_ref[...]
    k = cb.shape[0]
    norm = jnp.sqrt(jnp.sum(cb * cb, axis=1, keepdims=True))
    cbn = cb / (norm + 1e-12)
    pad = cbn_ref.shape[1] - cbn.shape[1]
    cbn_ref[...] = jnp.concatenate(
        [cbn, jnp.zeros((k, pad), jnp.float32)], axis=1)
    d = cb.shape[1]
    zb = z_ref[...].reshape(-1, d)
    znorm = jnp.sqrt(jnp.sum(zb * zb, axis=1, keepdims=True))
    zn = zb / (znorm + 1e-12)
    dist = jax.lax.dot_general(
        zn, cbn, dimension_numbers=(((1,), (1,)), ((), ())),
        preferred_element_type=jnp.float32)
    m = jnp.max(dist, axis=1, keepdims=True)
    hot = (dist == m).astype(jnp.float32)
    iota_col = jax.lax.broadcasted_iota(jnp.int32, (k, 1), 0)
    hi = (iota_col // 256).astype(jnp.float32)
    lo = (iota_col % 256).astype(jnp.float32)
    aug = jnp.concatenate([jnp.ones((k, 1), jnp.float32), hi, lo], axis=1)
    res = jnp.dot(hot, aug, preferred_element_type=jnp.float32)
    ind = res[:, 1] * 256.0 + res[:, 2]
    idx_ref[...] = ind.astype(jnp.int32).reshape(idx_ref.shape)
    cnt = res[:, 0]

    @pl.when(jnp.max(cnt) > 1.5)
    def _fixup():
        ind = jnp.argmax(dist, axis=1)
        idx_ref[...] = ind.reshape(idx_ref.shape)


def _tc_scores(z, codebook):
    b, s, d = z.shape
    rows = _TILE // s
    n = b * s
    grid = b // rows
    idx, cbn = pl.pallas_call(
        _score_body,
        grid=(grid,),
        in_specs=[
            pl.BlockSpec((rows, s, d), lambda i: (i, 0, 0)),
            pl.BlockSpec(codebook.shape, lambda i: (0, 0)),
        ],
        out_specs=[
            pl.BlockSpec((1, 1, _TILE), lambda i: (i, 0, 0)),
            pl.BlockSpec((codebook.shape[0], _DPAD), lambda i: (0, 0)),
        ],
        out_shape=[
            jax.ShapeDtypeStruct((grid, 1, _TILE), jnp.int32),
            jax.ShapeDtypeStruct((codebook.shape[0], _DPAD), jnp.float32),
        ],
        compiler_params=pltpu.CompilerParams(
            dimension_semantics=("arbitrary",)),
    )(z, codebook)
    return idx.reshape(n), cbn


def _make_sc_gather(n, dp):
    nw = _NC * _NS
    b_per_w = n // nw
    half = b_per_w // 2
    nchunks = half // _CHUNK
    mesh = plsc.VectorSubcoreMesh(
        core_axis_name="c", subcore_axis_name="s",
        num_cores=_NC, num_subcores=_NS)

    @functools.partial(
        pl.kernel, mesh=mesh,
        out_type=jax.ShapeDtypeStruct((n, dp), jnp.float32),
        scratch_types=[
            pltpu.VMEM((b_per_w,), jnp.int32),
            pltpu.VMEM((half, dp), jnp.float32),
            pltpu.SemaphoreType.DMA,
        ],
    )
    def sc_gather(table_hbm, idx_hbm, out_hbm, idx_v, rows_v, sem):
        wid = lax.axis_index("s") * _NC + lax.axis_index("c")
        base = wid * b_per_w
        pltpu.sync_copy(idx_hbm.at[pl.ds(base, b_per_w)], idx_v)
        for h in range(2):
            for c in range(nchunks):
                pltpu.async_copy(
                    table_hbm.at[idx_v.at[pl.ds((h * nchunks + c) * _CHUNK,
                                                _CHUNK)]],
                    rows_v.at[pl.ds(c * _CHUNK, _CHUNK)], sem).wait()
            pltpu.sync_copy(
                rows_v, out_hbm.at[pl.ds(base + h * half, half)])

    return sc_gather


def kernel(z, codebook):
    b, s, d = z.shape
    n = b * s
    idx, cbn_pad = _tc_scores(z, codebook)
    out = _make_sc_gather(n, _DPAD)(cbn_pad, idx)
    return out[:, :d].reshape(b, s, d)


# speedup vs baseline: 1.6744x; 1.6744x over previous
import jax
import jax.numpy as jnp
from jax.experimental import pallas as pl
from jax.experimental.pallas import tpu as pltpu


_TILE = 4096


def _vq_body(z_ref, cb_ref, out_ref):
    cb = cb_ref[...]
    k = cb.shape[0]
    norm = jnp.sqrt(jnp.sum(cb * cb, axis=1, keepdims=True))
    cbn = cb / (norm + 1e-12)
    d = cb.shape[1]
    zb = z_ref[...].reshape(-1, d)
    znorm = jnp.sqrt(jnp.sum(zb * zb, axis=1, keepdims=True))
    zn = zb / (znorm + 1e-12)
    dist = jax.lax.dot_general(
        zn, cbn, dimension_numbers=(((1,), (1,)), ((), ())),
        preferred_element_type=jnp.float32)
    m = jnp.max(dist, axis=1, keepdims=True)
    hot = (dist == m).astype(jnp.float32)
    aug = jnp.concatenate([cbn, jnp.ones((k, 1), jnp.float32)], axis=1)
    res = jnp.dot(hot, aug, preferred_element_type=jnp.float32)
    out_ref[...] = res[:, :-1].reshape(out_ref.shape)
    cnt = res[:, -1]

    @pl.when(jnp.max(cnt) > 1.5)
    def _fixup():
        ind = jnp.argmax(dist, axis=1)
        iota = jax.lax.broadcasted_iota(jnp.int32, dist.shape, 1)
        onehot = (iota == ind[:, None]).astype(jnp.float32)
        out_ref[...] = jnp.dot(
            onehot, cbn, preferred_element_type=jnp.float32
        ).reshape(out_ref.shape)


def kernel(z, codebook):
    b, s, d = z.shape
    rows = _TILE // s
    return pl.pallas_call(
        _vq_body,
        grid=(b // rows,),
        in_specs=[
            pl.BlockSpec((rows, s, d), lambda i: (i, 0, 0)),
            pl.BlockSpec(codebook.shape, lambda i: (0, 0)),
        ],
        out_specs=pl.BlockSpec((rows, s, d), lambda i: (i, 0, 0)),
        out_shape=jax.ShapeDtypeStruct((b, s, d), jnp.float32),
        compiler_params=pltpu.CompilerParams(
            dimension_semantics=("parallel",)),
    )(z, codebook)
